# baseline (speedup 1.0000x reference)
import functools

import jax
import jax.numpy as jnp
from jax import lax
from jax.experimental import pallas as pl
from jax.experimental.pallas import tpu as pltpu
from jax.experimental.pallas import tpu_sc as plsc

NUM_CORES = 2
NUM_SUBCORES = 16
NUM_WORKERS = NUM_CORES * NUM_SUBCORES
CHUNK = 128


def _embed_kernel(n_chunks_per_worker, d,
                  table_hbm, idx_hbm, out_hbm,
                  idx_v, rows0, rows1, sem0, sem1):
    cid = lax.axis_index("c")
    sid = lax.axis_index("s")
    wid = sid * NUM_CORES + cid
    base = wid * (n_chunks_per_worker * CHUNK)

    pltpu.sync_copy(idx_hbm.at[wid], idx_v)

    n2 = n_chunks_per_worker // 2

    def body(j, carry):
        c0 = 2 * j
        c1 = 2 * j + 1
        g0 = pltpu.async_copy(table_hbm.at[idx_v.at[c0]], rows0, sem0)
        g1 = pltpu.async_copy(table_hbm.at[idx_v.at[c1]], rows1, sem1)
        g0.wait()
        pltpu.sync_copy(rows0, out_hbm.at[pl.ds(base + c0 * CHUNK, CHUNK)])
        g1.wait()
        pltpu.sync_copy(rows1, out_hbm.at[pl.ds(base + c1 * CHUNK, CHUNK)])
        return carry

    lax.fori_loop(0, n2, body, 0)


def kernel(x, table):
    b_total = x.size
    d = table.shape[1]
    idx = x.reshape(-1).astype(jnp.int32)
    n_chunks_per_worker = b_total // (NUM_WORKERS * CHUNK)
    idx3 = idx.reshape(NUM_WORKERS, n_chunks_per_worker, CHUNK)

    mesh = plsc.VectorSubcoreMesh(core_axis_name="c", subcore_axis_name="s")
    out = pl.kernel(
        functools.partial(_embed_kernel, n_chunks_per_worker, d),
        out_type=jax.ShapeDtypeStruct((b_total, d), jnp.float32),
        mesh=mesh,
        scratch_types=[
            pltpu.VMEM((n_chunks_per_worker, CHUNK), jnp.int32),
            pltpu.VMEM((CHUNK, d), jnp.float32),
            pltpu.VMEM((CHUNK, d), jnp.float32),
            pltpu.SemaphoreType.DMA,
            pltpu.SemaphoreType.DMA,
        ],
        compiler_params=pltpu.CompilerParams(use_tc_tiling_on_sc=False),
    )(table, idx3)
    return out.reshape(*x.shape, d)


# speedup vs baseline: 1.0397x; 1.0397x over previous
import functools

import jax
import jax.numpy as jnp
from jax import lax
from jax.experimental import pallas as pl
from jax.experimental.pallas import tpu as pltpu
from jax.experimental.pallas import tpu_sc as plsc

NUM_CORES = 2
NUM_SUBCORES = 16
NUM_WORKERS = NUM_CORES * NUM_SUBCORES
CHUNK = 512


def _embed_kernel(n_chunks_per_worker, d,
                  table_hbm, idx_hbm, out_hbm,
                  idx_v, rows0, rows1, sem0, sem1):
    cid = lax.axis_index("c")
    sid = lax.axis_index("s")
    wid = sid * NUM_CORES + cid
    base = wid * (n_chunks_per_worker * CHUNK)

    pltpu.sync_copy(idx_hbm.at[wid], idx_v)

    n2 = n_chunks_per_worker // 2

    def body(j, carry):
        c0 = 2 * j
        c1 = 2 * j + 1
        g0 = pltpu.async_copy(table_hbm.at[idx_v.at[c0]], rows0, sem0)
        g1 = pltpu.async_copy(table_hbm.at[idx_v.at[c1]], rows1, sem1)
        g0.wait()
        pltpu.sync_copy(rows0, out_hbm.at[pl.ds(base + c0 * CHUNK, CHUNK)])
        g1.wait()
        pltpu.sync_copy(rows1, out_hbm.at[pl.ds(base + c1 * CHUNK, CHUNK)])
        return carry

    lax.fori_loop(0, n2, body, 0)


def kernel(x, table):
    b_total = x.size
    d = table.shape[1]
    idx = x.reshape(-1).astype(jnp.int32)
    n_chunks_per_worker = b_total // (NUM_WORKERS * CHUNK)
    idx3 = idx.reshape(NUM_WORKERS, n_chunks_per_worker, CHUNK)

    mesh = plsc.VectorSubcoreMesh(core_axis_name="c", subcore_axis_name="s")
    out = pl.kernel(
        functools.partial(_embed_kernel, n_chunks_per_worker, d),
        out_type=jax.ShapeDtypeStruct((b_total, d), jnp.float32),
        mesh=mesh,
        scratch_types=[
            pltpu.VMEM((n_chunks_per_worker, CHUNK), jnp.int32),
            pltpu.VMEM((CHUNK, d), jnp.float32),
            pltpu.VMEM((CHUNK, d), jnp.float32),
            pltpu.SemaphoreType.DMA,
            pltpu.SemaphoreType.DMA,
        ],
        compiler_params=pltpu.CompilerParams(use_tc_tiling_on_sc=False),
    )(table, idx3)
    return out.reshape(*x.shape, d)


# speedup vs baseline: 1.0451x; 1.0052x over previous
import functools

import jax
import jax.numpy as jnp
from jax import lax
from jax.experimental import pallas as pl
from jax.experimental.pallas import tpu as pltpu
from jax.experimental.pallas import tpu_sc as plsc

NUM_CORES = 2
NUM_SUBCORES = 16
NUM_WORKERS = NUM_CORES * NUM_SUBCORES
CHUNK = 128
K = 4


def _embed_kernel(n_chunks, d,
                  table_hbm, idx_hbm, out_hbm,
                  idx_v, a0, a1, a2, a3, b0, b1, b2, b3,
                  gsem_a, gsem_b, wsem_a, wsem_b):
    cid = lax.axis_index("c")
    sid = lax.axis_index("s")
    wid = sid * NUM_CORES + cid
    base = wid * (n_chunks * CHUNK)
    bufs_a = (a0, a1, a2, a3)
    bufs_b = (b0, b1, b2, b3)
    n_groups = n_chunks // K

    pltpu.sync_copy(idx_hbm.at[wid], idx_v)

    def fire_g(g, bufs, sem):
        for j in range(K):
            pltpu.async_copy(table_hbm.at[idx_v.at[g * K + j]], bufs[j], sem)

    def drain_g(g, bufs, sem):
        for j in range(K):
            pltpu.make_async_copy(
                table_hbm.at[idx_v.at[g * K + j]], bufs[j], sem).wait()

    def fire_w(g, bufs, sem):
        for j in range(K):
            c = g * K + j
            pltpu.async_copy(
                bufs[j], out_hbm.at[pl.ds(base + c * CHUNK, CHUNK)], sem)

    def drain_w(g, bufs, sem):
        for j in range(K):
            c = g * K + j
            pltpu.make_async_copy(
                bufs[j], out_hbm.at[pl.ds(base + c * CHUNK, CHUNK)],
                sem).wait()

    fire_g(0, bufs_a, gsem_a)
    drain_g(0, bufs_a, gsem_a)
    fire_w(0, bufs_a, wsem_a)
    fire_g(1, bufs_b, gsem_b)

    def body(i, carry):
        g0 = 2 * i + 1
        g1 = g0 + 1
        drain_g(g0, bufs_b, gsem_b)
        fire_w(g0, bufs_b, wsem_b)
        drain_w(g0 - 1, bufs_a, wsem_a)
        fire_g(g1, bufs_a, gsem_a)
        drain_g(g1, bufs_a, gsem_a)
        fire_w(g1, bufs_a, wsem_a)
        drain_w(g1 - 1, bufs_b, wsem_b)
        fire_g(g1 + 1, bufs_b, gsem_b)
        return carry

    lax.fori_loop(0, (n_groups - 2) // 2, body, 0)

    g_last = n_groups - 1
    drain_g(g_last, bufs_b, gsem_b)
    fire_w(g_last, bufs_b, wsem_b)
    drain_w(g_last - 1, bufs_a, wsem_a)
    drain_w(g_last, bufs_b, wsem_b)


def kernel(x, table):
    b_total = x.size
    d = table.shape[1]
    idx = x.reshape(-1).astype(jnp.int32)
    n_chunks = b_total // (NUM_WORKERS * CHUNK)
    assert b_total == NUM_WORKERS * n_chunks * CHUNK
    assert n_chunks % (2 * K) == 0
    idx3 = idx.reshape(NUM_WORKERS, n_chunks, CHUNK)

    mesh = plsc.VectorSubcoreMesh(core_axis_name="c", subcore_axis_name="s")
    rows_t = pltpu.VMEM((CHUNK, d), jnp.float32)
    out = pl.kernel(
        functools.partial(_embed_kernel, n_chunks, d),
        out_type=jax.ShapeDtypeStruct((b_total, d), jnp.float32),
        mesh=mesh,
        scratch_types=[pltpu.VMEM((n_chunks, CHUNK), jnp.int32)]
        + [rows_t] * 8
        + [pltpu.SemaphoreType.DMA] * 4,
        compiler_params=pltpu.CompilerParams(use_tc_tiling_on_sc=False),
    )(table, idx3)
    return out.reshape(*x.shape, d)
